# baseline (speedup 1.0000x reference)
import functools

import jax
import jax.numpy as jnp
from jax import lax
from jax.experimental import pallas as pl
from jax.experimental.pallas import tpu as pltpu
from jax.experimental.pallas import tpu_sc as plsc

NS = 16
C = 64
CM = 8
_EPS = 1e-5


def _proj_body(x_ref, wqt_ref, bq_ref, wkt_ref, bk_ref, wvt_ref, bv_ref,
               xq_ref, xk_ref, xv_ref):
    x = x_ref[...]
    xq_ref[...] = jnp.dot(x, wqt_ref[...], preferred_element_type=jnp.float32) + bq_ref[...]
    xk_ref[...] = jnp.dot(x, wkt_ref[...], preferred_element_type=jnp.float32) + bk_ref[...]
    xv_ref[...] = jnp.dot(x, wvt_ref[...], preferred_element_type=jnp.float32) + bv_ref[...]


def _proj(x, wqt, bq, wkt, bk, wvt, bv):
    n = x.shape[0]
    pb = min(2048, n)
    grid = n // pb
    f32 = jnp.float32
    blk = lambda shape: pl.BlockSpec(shape, lambda i: (0, 0))
    return pl.pallas_call(
        _proj_body,
        grid=(grid,),
        in_specs=[
            pl.BlockSpec((pb, C), lambda i: (i, 0)),
            blk((C, C)), blk((1, C)), blk((C, C)), blk((1, C)), blk((C, C)), blk((1, C)),
        ],
        out_specs=[pl.BlockSpec((pb, C), lambda i: (i, 0))] * 3,
        out_shape=[jax.ShapeDtypeStruct((n, C), f32)] * 3,
    )(x, wqt, bq, wkt, bk, wvt, bv)


def _thr_body(pt_ref, q_ref, dt_ref):
    n = pt_ref.shape[1]
    qb = q_ref.shape[0]
    cw = 128
    inf = jnp.float32(jnp.inf)
    pt = pt_ref[...]
    psq = jnp.sum(pt * pt, axis=0, keepdims=True)
    q = q_ref[...]
    qsq = jnp.sum(q * q, axis=1, keepdims=True)
    d = (qsq + psq) - 2.0 * jnp.dot(q, pt, preferred_element_type=jnp.float32)
    cm = jnp.min(d.reshape(qb, n // cw, cw), axis=2)
    m = None
    for _ in range(NS):
        m = jnp.min(cm, axis=1, keepdims=True)
        cm = jnp.where(cm == m, inf, cm)
    dt_ref[...] = jnp.reshape(jnp.where(d <= m, d, inf), (qb * (n // 128), 128))


def _thr(p8, pt):
    n = p8.shape[0]
    qb = min(128, n)
    return pl.pallas_call(
        _thr_body,
        grid=(n // qb,),
        in_specs=[
            pl.BlockSpec((8, n), lambda i: (0, 0)),
            pl.BlockSpec((qb, 8), lambda i: (i, 0)),
        ],
        out_specs=pl.BlockSpec((qb * (n // 128), 128), lambda i: (i, 0)),
        out_shape=jax.ShapeDtypeStruct((n * (n // 128), 128), jnp.float32),
    )(pt, p8)


def _sc_topk(dthr_flat, n):
    nw = 32
    rpw = n // nw
    grp = 4 if rpw % 4 == 0 else 1
    nv = n // 16
    mesh = plsc.VectorSubcoreMesh(core_axis_name="c", subcore_axis_name="s")
    inf = jnp.float32(jnp.inf)
    i32 = jnp.int32

    @functools.partial(
        pl.kernel,
        mesh=mesh,
        compiler_params=pltpu.CompilerParams(use_tc_tiling_on_sc=False,
                                             needs_layout_passes=False),
        out_type=jax.ShapeDtypeStruct((n * NS,), i32),
        scratch_types=[
            pltpu.VMEM((grp * n,), jnp.float32),
            pltpu.VMEM((n,), jnp.float32),
            pltpu.VMEM((n,), i32),
            pltpu.VMEM((rpw * NS,), i32),
        ],
    )
    def tk(d_hbm, idx_hbm, rows_v, cv_v, ci_v, out_v):
        wid = lax.axis_index("s") * 2 + lax.axis_index("c")
        base = wid * rpw
        lane = lax.iota(i32, 16)

        def row_fn(g, loc):
            unr = 16

            def scan_grp(jg, evt):
                vs = [rows_v[pl.ds(g * n + (jg * unr + u) * 16, 16)]
                      for u in range(unr)]
                ms = [v < inf for v in vs]
                anyv = ms[0]
                for u in range(1, unr):
                    anyv = anyv | ms[u]

                def do(evt):
                    for u in range(unr):
                        cnt = plsc.all_reduce_population_count(ms[u])
                        slots = evt * 16 + lane
                        plsc.store_scatter(cv_v, [slots], vs[u])
                        plsc.store_scatter(ci_v, [slots],
                                           (jg * unr + u) * 16 + lane)
                        evt = evt + jnp.minimum(cnt, 1)
                    return evt
                return lax.cond(jnp.any(anyv), do, lambda e: e, evt)
            evt = lax.fori_loop(0, nv // unr, scan_grp, jnp.zeros((16,), i32))
            nevt = jnp.max(evt)

            def merge(e, carry):
                rk, ri = carry
                k2 = cv_v[pl.ds(e * 16, 16)]
                i2 = ci_v[pl.ds(e * 16, 16)]
                k2s, i2s = plsc.sort_key_val(k2, i2)
                k2r = lax.rev(k2s, (0,))
                i2r = lax.rev(i2s, (0,))
                lo = (rk < k2r) | ((rk == k2r) & (ri < i2r))
                lk = jnp.where(lo, rk, k2r)
                li = jnp.where(lo, ri, i2r)
                ks, vs = plsc.sort_key_val(lk, li)
                return (ks, vs)
            rk0 = jnp.full((16,), inf, jnp.float32)
            ri0 = jnp.full((16,), n, i32)
            _, ri = lax.fori_loop(0, nevt, merge, (rk0, ri0))
            out_v[pl.ds(loc * NS, NS)] = ri

        def group_fn(gi, _):
            row0 = base + gi * grp
            pltpu.sync_copy(d_hbm.at[pl.ds(row0 * n, grp * n)], rows_v)
            for g in range(grp):
                row_fn(g, gi * grp + g)
            return 0
        lax.fori_loop(0, rpw // grp, group_fn, 0)
        pltpu.sync_copy(out_v, idx_hbm.at[pl.ds(base * NS, rpw * NS)])

    return tk(dthr_flat)


def _gather3(xk, xv, p16, idxf):
    m = idxf.shape[0]
    nw = 32
    rpw = m // nw
    gch = min(512, rpw)
    mesh = plsc.VectorSubcoreMesh(core_axis_name="c", subcore_axis_name="s")
    f32 = jnp.float32

    @functools.partial(
        pl.kernel,
        mesh=mesh,
        compiler_params=pltpu.CompilerParams(use_tc_tiling_on_sc=False),
        out_type=[
            jax.ShapeDtypeStruct((m, C), f32),
            jax.ShapeDtypeStruct((m, C), f32),
            jax.ShapeDtypeStruct((m, 16), f32),
        ],
        scratch_types=[
            pltpu.VMEM((gch,), jnp.int32),
            pltpu.VMEM((gch, C), f32),
            pltpu.VMEM((gch, C), f32),
            pltpu.VMEM((gch, 16), f32),
            pltpu.SemaphoreType.DMA,
        ],
    )
    def gk(xk_hbm, xv_hbm, p_hbm, idx_hbm, xkg_hbm, xvg_hbm, pg_hbm,
           idx_v, xk_v, xv_v, p_v, sem):
        wid = lax.axis_index("s") * 2 + lax.axis_index("c")
        base = wid * rpw
        for t in range(rpw // gch):
            off = base + t * gch
            pltpu.sync_copy(idx_hbm.at[pl.ds(off, gch)], idx_v)
            c1 = pltpu.async_copy(xk_hbm.at[idx_v], xk_v, sem)
            c2 = pltpu.async_copy(xv_hbm.at[idx_v], xv_v, sem)
            c3 = pltpu.async_copy(p_hbm.at[idx_v], p_v, sem)
            c1.wait()
            c2.wait()
            c3.wait()
            pltpu.sync_copy(xk_v, xkg_hbm.at[pl.ds(off, gch)])
            pltpu.sync_copy(xv_v, xvg_hbm.at[pl.ds(off, gch)])
            pltpu.sync_copy(p_v, pg_hbm.at[pl.ds(off, gch)])

    return gk(xk, xv, p16, idxf)


def _bn_coeffs(sums, gamma, beta, count):
    mean = sums[0] / count
    var = sums[1] / count - mean * mean
    scale = gamma / jnp.sqrt(var + _EPS)
    shift = beta - mean * scale
    return scale[None, :], shift[None, :]


def _h_block(pg_ref, pc_ref, w1_ref, b1_ref):
    b = pg_ref.shape[0]
    pr = pg_ref[...] - pc_ref[...][:, None, :]
    pr2 = pr.reshape(b * NS, 16)
    return jnp.dot(pr2, w1_ref[...], preferred_element_type=jnp.float32) + b1_ref[...]


def _w_block(h, xq_ref, xkg_ref, sc1_ref, sh1_ref, wp2t_ref, bp2_ref):
    b = xq_ref.shape[0]
    hn = jnp.maximum(h * sc1_ref[...] + sh1_ref[...], 0.0)
    pr2 = jnp.dot(hn, wp2t_ref[...], preferred_element_type=jnp.float32) + bp2_ref[...]
    xk2 = xkg_ref[...].reshape(b * NS, C)
    xqe = jnp.broadcast_to(xq_ref[...][:, None, :], (b, NS, C)).reshape(b * NS, C)
    return xk2 - xqe + pr2, pr2


def _acc_sums(out_ref, vals):
    @pl.when(pl.program_id(0) == 0)
    def _():
        out_ref[...] = jnp.zeros_like(out_ref)
    out_ref[0:1, :] += jnp.sum(vals, axis=0, keepdims=True)
    out_ref[1:2, :] += jnp.sum(vals * vals, axis=0, keepdims=True)


def _stats1_body(pg_ref, pc_ref, w1_ref, b1_ref, out_ref):
    _acc_sums(out_ref, _h_block(pg_ref, pc_ref, w1_ref, b1_ref))


def _stats2_body(pg_ref, pc_ref, w1_ref, b1_ref, xq_ref, xkg_ref,
                 sc1_ref, sh1_ref, wp2t_ref, bp2_ref, out_ref):
    h = _h_block(pg_ref, pc_ref, w1_ref, b1_ref)
    w, _ = _w_block(h, xq_ref, xkg_ref, sc1_ref, sh1_ref, wp2t_ref, bp2_ref)
    _acc_sums(out_ref, w)


def _stats3_body(pg_ref, pc_ref, w1_ref, b1_ref, xq_ref, xkg_ref,
                 sc1_ref, sh1_ref, wp2t_ref, bp2_ref, sc2_ref, sh2_ref,
                 ww1t_ref, bw1_ref, out_ref):
    h = _h_block(pg_ref, pc_ref, w1_ref, b1_ref)
    w, _ = _w_block(h, xq_ref, xkg_ref, sc1_ref, sh1_ref, wp2t_ref, bp2_ref)
    wn = jnp.maximum(w * sc2_ref[...] + sh2_ref[...], 0.0)
    u = jnp.dot(wn, ww1t_ref[...], preferred_element_type=jnp.float32) + bw1_ref[...]
    _acc_sums(out_ref, u)


def _final_body(pg_ref, pc_ref, w1_ref, b1_ref, xq_ref, xkg_ref,
                sc1_ref, sh1_ref, wp2t_ref, bp2_ref, sc2_ref, sh2_ref,
                ww1t_ref, bw1_ref, sc3_ref, sh3_ref, ww2t_ref, bw2_ref,
                xvg_ref, out_ref):
    b = xq_ref.shape[0]
    h = _h_block(pg_ref, pc_ref, w1_ref, b1_ref)
    w, pr2 = _w_block(h, xq_ref, xkg_ref, sc1_ref, sh1_ref, wp2t_ref, bp2_ref)
    wn = jnp.maximum(w * sc2_ref[...] + sh2_ref[...], 0.0)
    u = jnp.dot(wn, ww1t_ref[...], preferred_element_type=jnp.float32) + bw1_ref[...]
    un = jnp.maximum(u * sc3_ref[...] + sh3_ref[...], 0.0)
    w2 = jnp.dot(un, ww2t_ref[...], preferred_element_type=jnp.float32) + bw2_ref[...]
    w3 = w2.reshape(b, NS, CM)
    mx = jnp.max(w3, axis=1, keepdims=True)
    e = jnp.exp(w3 - mx)
    sm = e / jnp.sum(e, axis=1, keepdims=True)
    smt = jnp.concatenate([sm] * (C // CM), axis=2)
    v = xvg_ref[...] + pr2.reshape(b, NS, C)
    out_ref[...] = jnp.sum(v * smt, axis=1)


def _param_specs(shapes):
    return [pl.BlockSpec(s, lambda i: tuple(0 for _ in s)) for s in shapes]


def _stats_call(body, n, sb, out_w, args, extra_specs):
    grid = n // sb
    specs = [
        pl.BlockSpec((sb, NS, 16), lambda i: (i, 0, 0)),
        pl.BlockSpec((sb, 16), lambda i: (i, 0)),
        pl.BlockSpec((16, 16), lambda i: (0, 0)),
        pl.BlockSpec((1, 16), lambda i: (0, 0)),
    ] + extra_specs
    return pl.pallas_call(
        body,
        grid=(grid,),
        in_specs=specs,
        out_specs=pl.BlockSpec((8, out_w), lambda i: (0, 0)),
        out_shape=jax.ShapeDtypeStruct((8, out_w), jnp.float32),
    )(*args)


def kernel(p, x, o, Wq, bq, Wk, bk, Wv, bv, Wp1, bp1, gp, bp_bn, Wp2, bp2,
           gw1, bw1_bn, Ww1, bw1, gw2, bw2_bn, Ww2, bw2):
    f32 = jnp.float32
    n = p.shape[0]
    count = jnp.float32(n * NS)

    p8 = jnp.concatenate([p, jnp.zeros((n, 5), f32)], axis=1)
    pt = p8.T
    p16 = jnp.concatenate([p, jnp.zeros((n, 13), f32)], axis=1)
    w1p = jnp.zeros((16, 16), f32).at[:3, :3].set(Wp1.T)
    b1p = jnp.zeros((1, 16), f32).at[0, :3].set(bp1)
    gp16 = jnp.zeros((16,), f32).at[:3].set(gp)
    bp16 = jnp.zeros((16,), f32).at[:3].set(bp_bn)
    wp2t = jnp.zeros((16, C), f32).at[:3, :].set(Wp2.T)
    bp2r = bp2[None, :]

    xq, xkf, xvf = _proj(x, Wq.T, bq[None, :], Wk.T, bk[None, :], Wv.T, bv[None, :])

    dthr = _thr(p8, pt).reshape(-1)
    idxf = _sc_topk(dthr, n)

    xkg2, xvg2, pg2 = _gather3(xkf, xvf, p16, idxf)
    pg = pg2.reshape(n, NS, 16)
    xkg = xkg2.reshape(n, NS, C)
    xvg = xvg2.reshape(n, NS, C)

    sb = min(2048, n)
    base_args = (pg, p16, w1p, b1p)
    s1 = _stats_call(_stats1_body, n, sb, 16, base_args, [])
    sc1, sh1 = _bn_coeffs(s1, gp16, bp16, count)

    sb2 = min(1024, n)
    ext2_specs = [
        pl.BlockSpec((sb2, C), lambda i: (i, 0)),
        pl.BlockSpec((sb2, NS, C), lambda i: (i, 0, 0)),
    ] + _param_specs([(1, 16), (1, 16), (16, C), (1, C)])
    args2 = (pg, p16, w1p, b1p, xq, xkg, sc1, sh1, wp2t, bp2r)
    s2 = _stats_call(_stats2_body, n, sb2, C, args2, ext2_specs)
    sc2, sh2 = _bn_coeffs(s2, gw1, bw1_bn, count)

    ext3_specs = ext2_specs + _param_specs([(1, C), (1, C), (C, CM), (1, CM)])
    args3 = args2 + (sc2, sh2, Ww1.T, bw1[None, :])
    s3 = _stats_call(_stats3_body, n, sb2, CM, args3, ext3_specs)
    sc3, sh3 = _bn_coeffs(s3, gw2, bw2_bn, count)

    fb = min(512, n)
    fin_specs = [
        pl.BlockSpec((fb, NS, 16), lambda i: (i, 0, 0)),
        pl.BlockSpec((fb, 16), lambda i: (i, 0)),
        pl.BlockSpec((16, 16), lambda i: (0, 0)),
        pl.BlockSpec((1, 16), lambda i: (0, 0)),
        pl.BlockSpec((fb, C), lambda i: (i, 0)),
        pl.BlockSpec((fb, NS, C), lambda i: (i, 0, 0)),
    ] + _param_specs([(1, 16), (1, 16), (16, C), (1, C), (1, C), (1, C),
                      (C, CM), (1, CM), (1, CM), (1, CM), (CM, CM), (1, CM)]) + [
        pl.BlockSpec((fb, NS, C), lambda i: (i, 0, 0)),
    ]
    out = pl.pallas_call(
        _final_body,
        grid=(n // fb,),
        in_specs=fin_specs,
        out_specs=pl.BlockSpec((fb, C), lambda i: (i, 0)),
        out_shape=jax.ShapeDtypeStruct((n, C), f32),
    )(pg, p16, w1p, b1p, xq, xkg, sc1, sh1, wp2t, bp2r, sc2, sh2,
      Ww1.T, bw1[None, :], sc3, sh3, Ww2.T, bw2[None, :], xvg)
    return out


# speedup vs baseline: 2.6743x; 2.6743x over previous
import functools

import jax
import jax.numpy as jnp
from jax import lax
from jax.experimental import pallas as pl
from jax.experimental.pallas import tpu as pltpu
from jax.experimental.pallas import tpu_sc as plsc

NS = 16
C = 64
CM = 8
_EPS = 1e-5


def _proj_body(x_ref, wqt_ref, bq_ref, wkt_ref, bk_ref, wvt_ref, bv_ref,
               xq_ref, xk_ref, xv_ref):
    x = x_ref[...]
    xq_ref[...] = jnp.dot(x, wqt_ref[...], preferred_element_type=jnp.float32) + bq_ref[...]
    xk_ref[...] = jnp.dot(x, wkt_ref[...], preferred_element_type=jnp.float32) + bk_ref[...]
    xv_ref[...] = jnp.dot(x, wvt_ref[...], preferred_element_type=jnp.float32) + bv_ref[...]


def _proj(x, wqt, bq, wkt, bk, wvt, bv):
    n = x.shape[0]
    pb = min(2048, n)
    grid = n // pb
    f32 = jnp.float32
    blk = lambda shape: pl.BlockSpec(shape, lambda i: (0, 0))
    return pl.pallas_call(
        _proj_body,
        grid=(grid,),
        in_specs=[
            pl.BlockSpec((pb, C), lambda i: (i, 0)),
            blk((C, C)), blk((1, C)), blk((C, C)), blk((1, C)), blk((C, C)), blk((1, C)),
        ],
        out_specs=[pl.BlockSpec((pb, C), lambda i: (i, 0))] * 3,
        out_shape=[jax.ShapeDtypeStruct((n, C), f32)] * 3,
    )(x, wqt, bq, wkt, bk, wvt, bv)


def _thr_body(pt_ref, q_ref, dt_ref, d_ref):
    n = pt_ref.shape[1]
    qb = q_ref.shape[0]
    cw = 128
    inf = jnp.float32(jnp.inf)
    pt = pt_ref[...]
    psq = jnp.sum(pt * pt, axis=0, keepdims=True)
    q = q_ref[...]
    qsq = jnp.sum(q * q, axis=1, keepdims=True)
    d_ref[...] = (qsq + psq) - 2.0 * jnp.dot(q, pt, preferred_element_type=jnp.float32)
    mins = [jnp.min(d_ref[:, c * cw:(c + 1) * cw], axis=1, keepdims=True)
            for c in range(n // cw)]
    cm = jnp.concatenate(mins, axis=1)
    m = None
    for _ in range(NS):
        m = jnp.min(cm, axis=1, keepdims=True)
        cm = jnp.where(cm == m, inf, cm)
    d = d_ref[...]
    dt_ref[...] = jnp.reshape(jnp.where(d <= m, d, inf), (qb * (n // 128), 128))


def _thr(p8, pt):
    n = p8.shape[0]
    qb = min(256, n)
    return pl.pallas_call(
        _thr_body,
        grid=(n // qb,),
        in_specs=[
            pl.BlockSpec((8, n), lambda i: (0, 0)),
            pl.BlockSpec((qb, 8), lambda i: (i, 0)),
        ],
        out_specs=pl.BlockSpec((qb * (n // 128), 128), lambda i: (i, 0)),
        out_shape=jax.ShapeDtypeStruct((n * (n // 128), 128), jnp.float32),
        scratch_shapes=[pltpu.VMEM((qb, n), jnp.float32)],
    )(pt, p8)


def _sc_topk(dthr_flat, n):
    nw = 32
    rpw = n // nw
    grp = 4 if rpw % 4 == 0 else 1
    nv = n // 16
    mesh = plsc.VectorSubcoreMesh(core_axis_name="c", subcore_axis_name="s")
    inf = jnp.float32(jnp.inf)
    i32 = jnp.int32

    @functools.partial(
        pl.kernel,
        mesh=mesh,
        compiler_params=pltpu.CompilerParams(use_tc_tiling_on_sc=False,
                                             needs_layout_passes=False),
        out_type=jax.ShapeDtypeStruct((n * NS,), i32),
        scratch_types=[
            pltpu.VMEM((grp * n,), jnp.float32),
            pltpu.VMEM((n,), jnp.float32),
            pltpu.VMEM((n,), i32),
            pltpu.VMEM((rpw * NS,), i32),
        ],
    )
    def tk(d_hbm, idx_hbm, rows_v, cv_v, ci_v, out_v):
        wid = lax.axis_index("s") * 2 + lax.axis_index("c")
        base = wid * rpw
        lane = lax.iota(i32, 16)

        def row_fn(g, loc):
            unr = 16

            def scan_grp(jg, evt):
                vs = [rows_v[pl.ds(g * n + (jg * unr + u) * 16, 16)]
                      for u in range(unr)]
                ms = [v < inf for v in vs]
                anyv = ms[0]
                for u in range(1, unr):
                    anyv = anyv | ms[u]

                def do(evt):
                    for u in range(unr):
                        cnt = plsc.all_reduce_population_count(ms[u])
                        slots = evt * 16 + lane
                        plsc.store_scatter(cv_v, [slots], vs[u])
                        plsc.store_scatter(ci_v, [slots],
                                           (jg * unr + u) * 16 + lane)
                        evt = evt + jnp.minimum(cnt, 1)
                    return evt
                return lax.cond(jnp.any(anyv), do, lambda e: e, evt)
            evt = lax.fori_loop(0, nv // unr, scan_grp, jnp.zeros((16,), i32))
            nevt = jnp.max(evt)

            def merge(e, carry):
                rk, ri = carry
                k2 = cv_v[pl.ds(e * 16, 16)]
                i2 = ci_v[pl.ds(e * 16, 16)]
                k2s, i2s = plsc.sort_key_val(k2, i2)
                k2r = lax.rev(k2s, (0,))
                i2r = lax.rev(i2s, (0,))
                lo = (rk < k2r) | ((rk == k2r) & (ri < i2r))
                lk = jnp.where(lo, rk, k2r)
                li = jnp.where(lo, ri, i2r)
                ks, vs = plsc.sort_key_val(lk, li)
                return (ks, vs)
            rk0 = jnp.full((16,), inf, jnp.float32)
            ri0 = jnp.full((16,), n, i32)
            _, ri = lax.fori_loop(0, nevt, merge, (rk0, ri0))
            out_v[pl.ds(loc * NS, NS)] = ri

        def group_fn(gi, _):
            row0 = base + gi * grp
            pltpu.sync_copy(d_hbm.at[pl.ds(row0 * n, grp * n)], rows_v)
            for g in range(grp):
                row_fn(g, gi * grp + g)
            return 0
        lax.fori_loop(0, rpw // grp, group_fn, 0)
        pltpu.sync_copy(out_v, idx_hbm.at[pl.ds(base * NS, rpw * NS)])

    return tk(dthr_flat)


def _gather3(xk, xv, p16, idxf):
    m = idxf.shape[0]
    nw = 32
    rpw = m // nw
    gch = min(512, rpw)
    mesh = plsc.VectorSubcoreMesh(core_axis_name="c", subcore_axis_name="s")
    f32 = jnp.float32

    @functools.partial(
        pl.kernel,
        mesh=mesh,
        compiler_params=pltpu.CompilerParams(use_tc_tiling_on_sc=False),
        out_type=[
            jax.ShapeDtypeStruct((m, C), f32),
            jax.ShapeDtypeStruct((m, C), f32),
            jax.ShapeDtypeStruct((m, 16), f32),
        ],
        scratch_types=[
            pltpu.VMEM((gch,), jnp.int32),
            pltpu.VMEM((gch, C), f32),
            pltpu.VMEM((gch, C), f32),
            pltpu.VMEM((gch, 16), f32),
            pltpu.SemaphoreType.DMA,
        ],
    )
    def gk(xk_hbm, xv_hbm, p_hbm, idx_hbm, xkg_hbm, xvg_hbm, pg_hbm,
           idx_v, xk_v, xv_v, p_v, sem):
        wid = lax.axis_index("s") * 2 + lax.axis_index("c")
        base = wid * rpw
        for t in range(rpw // gch):
            off = base + t * gch
            pltpu.sync_copy(idx_hbm.at[pl.ds(off, gch)], idx_v)
            c1 = pltpu.async_copy(xk_hbm.at[idx_v], xk_v, sem)
            c2 = pltpu.async_copy(xv_hbm.at[idx_v], xv_v, sem)
            c3 = pltpu.async_copy(p_hbm.at[idx_v], p_v, sem)
            c1.wait()
            c2.wait()
            c3.wait()
            pltpu.sync_copy(xk_v, xkg_hbm.at[pl.ds(off, gch)])
            pltpu.sync_copy(xv_v, xvg_hbm.at[pl.ds(off, gch)])
            pltpu.sync_copy(p_v, pg_hbm.at[pl.ds(off, gch)])

    return gk(xk, xv, p16, idxf)


def _bn_coeffs(sums, gamma, beta, count):
    mean = sums[0] / count
    var = sums[1] / count - mean * mean
    scale = gamma / jnp.sqrt(var + _EPS)
    shift = beta - mean * scale
    return scale[None, :], shift[None, :]


def _h_block(pg_ref, pc_ref, w1_ref, b1_ref):
    b = pg_ref.shape[0]
    pr = pg_ref[...] - pc_ref[...][:, None, :]
    pr2 = pr.reshape(b * NS, 16)
    return jnp.dot(pr2, w1_ref[...], preferred_element_type=jnp.float32) + b1_ref[...]


def _w_block(h, xq_ref, xkg_ref, sc1_ref, sh1_ref, wp2t_ref, bp2_ref):
    b = xq_ref.shape[0]
    hn = jnp.maximum(h * sc1_ref[...] + sh1_ref[...], 0.0)
    pr2 = jnp.dot(hn, wp2t_ref[...], preferred_element_type=jnp.float32) + bp2_ref[...]
    xk2 = xkg_ref[...].reshape(b * NS, C)
    xqe = jnp.broadcast_to(xq_ref[...][:, None, :], (b, NS, C)).reshape(b * NS, C)
    return xk2 - xqe + pr2, pr2


def _acc_sums(out_ref, vals):
    @pl.when(pl.program_id(0) == 0)
    def _():
        out_ref[...] = jnp.zeros_like(out_ref)
    out_ref[0:1, :] += jnp.sum(vals, axis=0, keepdims=True)
    out_ref[1:2, :] += jnp.sum(vals * vals, axis=0, keepdims=True)


def _stats1_body(pg_ref, pc_ref, w1_ref, b1_ref, out_ref):
    _acc_sums(out_ref, _h_block(pg_ref, pc_ref, w1_ref, b1_ref))


def _stats2_body(pg_ref, pc_ref, w1_ref, b1_ref, xq_ref, xkg_ref,
                 sc1_ref, sh1_ref, wp2t_ref, bp2_ref, out_ref):
    h = _h_block(pg_ref, pc_ref, w1_ref, b1_ref)
    w, _ = _w_block(h, xq_ref, xkg_ref, sc1_ref, sh1_ref, wp2t_ref, bp2_ref)
    _acc_sums(out_ref, w)


def _stats3_body(pg_ref, pc_ref, w1_ref, b1_ref, xq_ref, xkg_ref,
                 sc1_ref, sh1_ref, wp2t_ref, bp2_ref, sc2_ref, sh2_ref,
                 ww1t_ref, bw1_ref, out_ref):
    h = _h_block(pg_ref, pc_ref, w1_ref, b1_ref)
    w, _ = _w_block(h, xq_ref, xkg_ref, sc1_ref, sh1_ref, wp2t_ref, bp2_ref)
    wn = jnp.maximum(w * sc2_ref[...] + sh2_ref[...], 0.0)
    u = jnp.dot(wn, ww1t_ref[...], preferred_element_type=jnp.float32) + bw1_ref[...]
    _acc_sums(out_ref, u)


def _final_body(pg_ref, pc_ref, w1_ref, b1_ref, xq_ref, xkg_ref,
                sc1_ref, sh1_ref, wp2t_ref, bp2_ref, sc2_ref, sh2_ref,
                ww1t_ref, bw1_ref, sc3_ref, sh3_ref, ww2t_ref, bw2_ref,
                xvg_ref, out_ref):
    b = xq_ref.shape[0]
    h = _h_block(pg_ref, pc_ref, w1_ref, b1_ref)
    w, pr2 = _w_block(h, xq_ref, xkg_ref, sc1_ref, sh1_ref, wp2t_ref, bp2_ref)
    wn = jnp.maximum(w * sc2_ref[...] + sh2_ref[...], 0.0)
    u = jnp.dot(wn, ww1t_ref[...], preferred_element_type=jnp.float32) + bw1_ref[...]
    un = jnp.maximum(u * sc3_ref[...] + sh3_ref[...], 0.0)
    w2 = jnp.dot(un, ww2t_ref[...], preferred_element_type=jnp.float32) + bw2_ref[...]
    w3 = w2.reshape(b, NS, CM)
    mx = jnp.max(w3, axis=1, keepdims=True)
    e = jnp.exp(w3 - mx)
    sm = e / jnp.sum(e, axis=1, keepdims=True)
    smt = jnp.concatenate([sm] * (C // CM), axis=2)
    v = xvg_ref[...] + pr2.reshape(b, NS, C)
    out_ref[...] = jnp.sum(v * smt, axis=1)


def _param_specs(shapes):
    return [pl.BlockSpec(s, lambda i: tuple(0 for _ in s)) for s in shapes]


def _stats_call(body, n, sb, out_w, args, extra_specs):
    grid = n // sb
    specs = [
        pl.BlockSpec((sb, NS, 16), lambda i: (i, 0, 0)),
        pl.BlockSpec((sb, 16), lambda i: (i, 0)),
        pl.BlockSpec((16, 16), lambda i: (0, 0)),
        pl.BlockSpec((1, 16), lambda i: (0, 0)),
    ] + extra_specs
    return pl.pallas_call(
        body,
        grid=(grid,),
        in_specs=specs,
        out_specs=pl.BlockSpec((8, out_w), lambda i: (0, 0)),
        out_shape=jax.ShapeDtypeStruct((8, out_w), jnp.float32),
    )(*args)


def kernel(p, x, o, Wq, bq, Wk, bk, Wv, bv, Wp1, bp1, gp, bp_bn, Wp2, bp2,
           gw1, bw1_bn, Ww1, bw1, gw2, bw2_bn, Ww2, bw2):
    f32 = jnp.float32
    n = p.shape[0]
    count = jnp.float32(n * NS)

    p8 = jnp.concatenate([p, jnp.zeros((n, 5), f32)], axis=1)
    pt = p8.T
    p16 = jnp.concatenate([p, jnp.zeros((n, 13), f32)], axis=1)
    w1p = jnp.zeros((16, 16), f32).at[:3, :3].set(Wp1.T)
    b1p = jnp.zeros((1, 16), f32).at[0, :3].set(bp1)
    gp16 = jnp.zeros((16,), f32).at[:3].set(gp)
    bp16 = jnp.zeros((16,), f32).at[:3].set(bp_bn)
    wp2t = jnp.zeros((16, C), f32).at[:3, :].set(Wp2.T)
    bp2r = bp2[None, :]

    xq, xkf, xvf = _proj(x, Wq.T, bq[None, :], Wk.T, bk[None, :], Wv.T, bv[None, :])

    dthr = _thr(p8, pt).reshape(-1)
    idxf = _sc_topk(dthr, n)

    xkg2, xvg2, pg2 = _gather3(xkf, xvf, p16, idxf)
    pg = pg2.reshape(n, NS, 16)
    xkg = xkg2.reshape(n, NS, C)
    xvg = xvg2.reshape(n, NS, C)

    sb = min(2048, n)
    base_args = (pg, p16, w1p, b1p)
    s1 = _stats_call(_stats1_body, n, sb, 16, base_args, [])
    sc1, sh1 = _bn_coeffs(s1, gp16, bp16, count)

    sb2 = min(1024, n)
    ext2_specs = [
        pl.BlockSpec((sb2, C), lambda i: (i, 0)),
        pl.BlockSpec((sb2, NS, C), lambda i: (i, 0, 0)),
    ] + _param_specs([(1, 16), (1, 16), (16, C), (1, C)])
    args2 = (pg, p16, w1p, b1p, xq, xkg, sc1, sh1, wp2t, bp2r)
    s2 = _stats_call(_stats2_body, n, sb2, C, args2, ext2_specs)
    sc2, sh2 = _bn_coeffs(s2, gw1, bw1_bn, count)

    ext3_specs = ext2_specs + _param_specs([(1, C), (1, C), (C, CM), (1, CM)])
    args3 = args2 + (sc2, sh2, Ww1.T, bw1[None, :])
    s3 = _stats_call(_stats3_body, n, sb2, CM, args3, ext3_specs)
    sc3, sh3 = _bn_coeffs(s3, gw2, bw2_bn, count)

    fb = min(512, n)
    fin_specs = [
        pl.BlockSpec((fb, NS, 16), lambda i: (i, 0, 0)),
        pl.BlockSpec((fb, 16), lambda i: (i, 0)),
        pl.BlockSpec((16, 16), lambda i: (0, 0)),
        pl.BlockSpec((1, 16), lambda i: (0, 0)),
        pl.BlockSpec((fb, C), lambda i: (i, 0)),
        pl.BlockSpec((fb, NS, C), lambda i: (i, 0, 0)),
    ] + _param_specs([(1, 16), (1, 16), (16, C), (1, C), (1, C), (1, C),
                      (C, CM), (1, CM), (1, CM), (1, CM), (CM, CM), (1, CM)]) + [
        pl.BlockSpec((fb, NS, C), lambda i: (i, 0, 0)),
    ]
    out = pl.pallas_call(
        _final_body,
        grid=(n // fb,),
        in_specs=fin_specs,
        out_specs=pl.BlockSpec((fb, C), lambda i: (i, 0)),
        out_shape=jax.ShapeDtypeStruct((n, C), f32),
    )(pg, p16, w1p, b1p, xq, xkg, sc1, sh1, wp2t, bp2r, sc2, sh2,
      Ww1.T, bw1[None, :], sc3, sh3, Ww2.T, bw2[None, :], xvg)
    return out
